# Initial kernel scaffold; baseline (speedup 1.0000x reference)
import jax
import jax.numpy as jnp
from jax.experimental import pallas as pl


def kernel(node_inp, node_type, edge_index, edge_type, edge_time, Wk, bk, Wq, bq, Wv, bv, rel_pri, rel_att, rel_msg, WMk, Wak, Wquery, bquery, Wkey, bkey):
    raise NotImplementedError("write your pallas kernel here")


# speedup vs baseline: 10.4075x; 10.4075x over previous
import functools
import math

import jax
import jax.numpy as jnp
from jax import lax
from jax.experimental import pallas as pl
from jax.experimental.pallas import tpu as pltpu
from jax.experimental.pallas import tpu_sc as plsc

_NC = 2
_NS = 16
_NW = _NC * _NS


def _block_diag(w):
    h, d, _ = w.shape
    eye = jnp.eye(h, dtype=w.dtype)
    out = jnp.einsum('hij,hg->higj', w, eye).reshape(h * d, h * d)
    return out


def _p1_body(x_ref, t_ref,
             wq0, wq1, bq0, bq1,
             wk0, wk1, bk0, bk1,
             wv0, wv1, bv0, bv1,
             bda0, bda1, bdm0, bdm1,
             q_out, k2_out, v2_out):
    x = x_ref[...]
    m0 = t_ref[...] == 0.0
    dot = functools.partial(jnp.dot, preferred_element_type=jnp.float32)
    q = jnp.where(m0, dot(x, wq0[...]) + bq0[...], dot(x, wq1[...]) + bq1[...])
    k = jnp.where(m0, dot(x, wk0[...]) + bk0[...], dot(x, wk1[...]) + bk1[...])
    v = jnp.where(m0, dot(x, wv0[...]) + bv0[...], dot(x, wv1[...]) + bv1[...])
    q_out[...] = q
    k2_out[:, 0, :] = dot(k, bda0[...])
    k2_out[:, 1, :] = dot(k, bda1[...])
    v2_out[:, 0, :] = dot(v, bdm0[...])
    v2_out[:, 1, :] = dot(v, bdm1[...])


def _run_p1(x, tf, wq, bq, wk, bk, wv, bv, bda, bdm):
    n, d = x.shape
    blk = 400
    grid = n // blk
    full = pl.BlockSpec((d, d), lambda i: (0, 0))
    fullb = pl.BlockSpec((1, d), lambda i: (0, 0))
    return pl.pallas_call(
        _p1_body,
        grid=(grid,),
        in_specs=[
            pl.BlockSpec((blk, d), lambda i: (i, 0)),
            pl.BlockSpec((blk, 1), lambda i: (i, 0)),
            full, full, fullb, fullb,
            full, full, fullb, fullb,
            full, full, fullb, fullb,
            full, full, full, full,
        ],
        out_specs=[
            pl.BlockSpec((blk, d), lambda i: (i, 0)),
            pl.BlockSpec((blk, 2, d), lambda i: (i, 0, 0)),
            pl.BlockSpec((blk, 2, d), lambda i: (i, 0, 0)),
        ],
        out_shape=[
            jax.ShapeDtypeStruct((n, d), jnp.float32),
            jax.ShapeDtypeStruct((n, 2, d), jnp.float32),
            jax.ShapeDtypeStruct((n, 2, d), jnp.float32),
        ],
    )(x, tf,
      wq[0], wq[1], bq[0][None], bq[1][None],
      wk[0], wk[1], bk[0][None], bk[1][None],
      wv[0], wv[1], bv[0][None], bv[1][None],
      bda[0], bda[1], bdm[0], bdm[1])


def _sc_gather(qh, k2h, v2h, dst, k2i, mi, e, np_, c, mr):
    ew = e // _NW
    nch = ew // c
    mesh = plsc.VectorSubcoreMesh(core_axis_name="c", subcore_axis_name="s")
    d = qh.shape[1]

    @functools.partial(
        pl.kernel, mesh=mesh,
        out_type=[
            jax.ShapeDtypeStruct((e, d), jnp.float32),
            jax.ShapeDtypeStruct((e, d), jnp.float32),
            jax.ShapeDtypeStruct((np_, d), jnp.float32),
        ],
        scratch_types=[
            pltpu.VMEM((c,), jnp.int32),
            pltpu.VMEM((c,), jnp.int32),
            pltpu.VMEM((c, d), jnp.float32),
            pltpu.VMEM((c, d), jnp.float32),
            pltpu.VMEM((mr,), jnp.int32),
            pltpu.VMEM((mr, d), jnp.float32),
            pltpu.SemaphoreType.DMA,
        ],
    )
    def k(qh_h, k2h_h, v2h_h, dst_h, k2i_h, mi_h, qg_o, kg_o, mg_o,
          dstb, k2b, bq, bk2, mib, mb, sem):
        wid = lax.axis_index("s") * _NC + lax.axis_index("c")

        def chunk(i, carry):
            base = wid * ew + i * c
            pltpu.sync_copy(dst_h.at[pl.ds(base, c)], dstb)
            pltpu.sync_copy(k2i_h.at[pl.ds(base, c)], k2b)
            pltpu.async_copy(qh_h.at[dstb], bq, sem).wait()
            pltpu.async_copy(k2h_h.at[k2b], bk2, sem).wait()
            pltpu.sync_copy(bq, qg_o.at[pl.ds(base, c)])
            pltpu.sync_copy(bk2, kg_o.at[pl.ds(base, c)])
            return carry

        lax.fori_loop(0, nch, chunk, 0)
        mbase = wid * mr
        pltpu.sync_copy(mi_h.at[pl.ds(mbase, mr)], mib)
        pltpu.async_copy(v2h_h.at[mib], mb, sem).wait()
        pltpu.sync_copy(mb, mg_o.at[pl.ds(mbase, mr)])

    return k(qh, k2h, v2h, dst, k2i, mi)


def _p2_body(qg_ref, kg_ref, out_ref):
    q = qg_ref[...]
    kk = kg_ref[...]
    p = q * kk
    b = p.shape[0]
    cols = [jnp.sum(p[:, h * 16:(h + 1) * 16], axis=1, keepdims=True)
            for h in range(8)]
    ex = jnp.exp(jnp.concatenate(cols, axis=1))
    ones = jnp.ones((b, 1), jnp.float32)
    zeros = jnp.zeros((b, 119), jnp.float32)
    out_ref[...] = jnp.concatenate([ex, ones, zeros], axis=1)


def _run_p2(qg, kg):
    e, d = qg.shape
    blk = 2000
    grid = e // blk
    return pl.pallas_call(
        _p2_body,
        grid=(grid,),
        in_specs=[
            pl.BlockSpec((blk, d), lambda i: (i, 0)),
            pl.BlockSpec((blk, d), lambda i: (i, 0)),
        ],
        out_specs=pl.BlockSpec((blk, d), lambda i: (i, 0)),
        out_shape=jax.ShapeDtypeStruct((e, d), jnp.float32),
    )(qg, kg)


def _sc_scatter(rows, dst, zeros_init, n, e, c):
    ew = e // _NW
    nch = ew // c
    mesh = plsc.VectorSubcoreMesh(core_axis_name="c", subcore_axis_name="s")

    @functools.partial(
        pl.kernel, mesh=mesh,
        out_type=jax.ShapeDtypeStruct((2, n, 128), jnp.float32),
        scratch_types=[
            pltpu.VMEM_SHARED((n, 128), jnp.float32),
            pltpu.VMEM((c, 128), jnp.float32),
            pltpu.VMEM((c,), jnp.int32),
            pltpu.SemaphoreType.DMA,
        ],
    )
    def k(rows_h, dst_h, zero_h, part_o, acc, rb, db, sem):
        cid = lax.axis_index("c")
        sid = lax.axis_index("s")

        @pl.when(sid == 0)
        def _():
            pltpu.sync_copy(zero_h, acc)

        plsc.subcore_barrier()
        wid = sid * _NC + cid

        def chunk(i, carry):
            base = wid * ew + i * c
            pltpu.sync_copy(rows_h.at[pl.ds(base, c)], rb)
            pltpu.sync_copy(dst_h.at[pl.ds(base, c)], db)
            pltpu.sync_copy(rb, acc.at[db], add=True)
            return carry

        lax.fori_loop(0, nch, chunk, 0)
        plsc.subcore_barrier()

        @pl.when(sid == 0)
        def _():
            pltpu.sync_copy(acc, part_o.at[cid])

    return k(rows, dst, zeros_init)


def _sc_seg_gather(p0, p1, dstn, np_, mr):
    mesh = plsc.VectorSubcoreMesh(core_axis_name="c", subcore_axis_name="s")

    @functools.partial(
        pl.kernel, mesh=mesh,
        out_type=jax.ShapeDtypeStruct((np_, 128), jnp.float32),
        scratch_types=[
            pltpu.VMEM((mr,), jnp.int32),
            pltpu.VMEM((mr, 128), jnp.float32),
            pltpu.VMEM((mr, 128), jnp.float32),
            pltpu.SemaphoreType.DMA,
        ],
    )
    def k(p0_h, p1_h, dstn_h, seg_o, ib, b0, b1, sem):
        wid = lax.axis_index("s") * _NC + lax.axis_index("c")
        base = wid * mr
        pltpu.sync_copy(dstn_h.at[pl.ds(base, mr)], ib)
        pltpu.async_copy(p0_h.at[ib], b0, sem).wait()
        pltpu.async_copy(p1_h.at[ib], b1, sem).wait()

        def add1(i, carry):
            b0[i, :] = b0[i, :] + b1[i, :]
            return carry

        lax.fori_loop(0, mr, add1, 0)
        pltpu.sync_copy(b0, seg_o.at[pl.ds(base, mr)])

    return k(p0, p1, dstn)


def _p3_body(ex_ref, sg_ref, p0_ref, p1_ref, msg_ref, x_ref,
             wm0, wm1, wm2, wa0, wa1, wa2, wq, bq, wkey, bkey, out_ref):
    dot = functools.partial(jnp.dot, preferred_element_type=jnp.float32)
    ex = ex_ref[...][:, 0:8]
    ss = sg_ref[...][:, 0:8]
    att = ex / (ss + 1e-16)
    deg = p0_ref[...][:, 8:9] + p1_ref[...][:, 8:9]
    m = msg_ref[...]
    b = m.shape[0]
    att128 = jnp.concatenate(
        [jnp.broadcast_to(att[:, h:h + 1], (b, 16)) for h in range(8)], axis=1)
    r1 = m * att128
    r2 = m * m * att128
    r3 = m * m * m * att128
    agg1 = deg * r1
    agg2 = deg * r2
    agg3 = deg * r3
    agg3 = jnp.sign(agg3) * jnp.exp(
        jnp.log(jnp.abs(agg3) + 1e-18) * (1.0 / 3.0))
    g1 = dot(agg1, wm0[...])
    g2 = dot(agg2, wm1[...])
    g3 = dot(agg3, wm2[...])
    qn = dot(x_ref[...], wq[...]) + bq[...]
    res = jnp.zeros((b, m.shape[1]), jnp.float32)
    for g, wa in ((g1, wa0), (g2, wa1), (g3, wa2)):
        front = dot(qn, wa[...])
        tail = dot(g, wkey[...]) + bkey[...]
        s = jnp.sum(front * tail, axis=1, keepdims=True)
        score = 1.0 / (1.0 + jnp.exp(-s))
        res = res + score * g
    out_ref[...] = res


def _run_p3(exn, segn, p0, p1, msgn, x, wmk, wak, wq, bq, wkey, bkey):
    n, d = x.shape
    blk = 400
    grid = n // blk
    full = pl.BlockSpec((d, d), lambda i: (0, 0))
    fullb = pl.BlockSpec((1, d), lambda i: (0, 0))
    row16 = pl.BlockSpec((blk, d), lambda i: (i, 0))
    return pl.pallas_call(
        _p3_body,
        grid=(grid,),
        in_specs=[
            row16, row16, row16, row16,
            pl.BlockSpec((blk, d), lambda i: (i, 0)),
            pl.BlockSpec((blk, d), lambda i: (i, 0)),
            full, full, full, full, full, full,
            full, fullb, full, fullb,
        ],
        out_specs=pl.BlockSpec((blk, d), lambda i: (i, 0)),
        out_shape=jax.ShapeDtypeStruct((n, d), jnp.float32),
    )(exn, segn, p0, p1, msgn, x,
      wmk[0], wmk[1], wmk[2], wak[0], wak[1], wak[2],
      wq, bq[None], wkey, bkey[None])


def kernel(node_inp, node_type, edge_index, edge_type, edge_time,
           Wk, bk, Wq, bq, Wv, bv, rel_pri, rel_att, rel_msg,
           WMk, Wak, Wquery, bquery, Wkey, bkey):
    n, in_dim = node_inp.shape
    e = edge_index.shape[1]
    num_rel, n_heads, d_k, _ = rel_att.shape
    sqrt_dk = math.sqrt(d_k)

    src = edge_index[0].astype(jnp.int32)
    dst = edge_index[1].astype(jnp.int32)
    et = edge_type.astype(jnp.int32)
    k2idx = src * num_rel + et

    tf = node_type.astype(jnp.float32).reshape(n, 1)
    bda = [_block_diag(rel_att[r] * (rel_pri[r][:, None, None] / sqrt_dk))
           for r in range(num_rel)]
    bdm = [_block_diag(rel_msg[r]) for r in range(num_rel)]

    qh, k2, v2 = _run_p1(node_inp, tf, Wq, bq, Wk, bk, Wv, bv, bda, bdm)
    k2 = k2.reshape(n * 2, in_dim)
    v2 = v2.reshape(n * 2, in_dim)

    mr = ((n + _NW - 1) // _NW + 7) // 8 * 8
    np_ = _NW * mr
    pad = np_ - n
    mi = jnp.concatenate([k2idx[:n], jnp.zeros((pad,), jnp.int32)])
    dstn = jnp.concatenate([dst[:n], jnp.zeros((pad,), jnp.int32)])

    qg, kg, msgg = _sc_gather(qh, k2, v2, dst, k2idx, mi, e, np_, 200, mr)
    rows = _run_p2(qg, kg)
    zeros_init = jnp.zeros((n, 128), jnp.float32)
    partials = _sc_scatter(rows, dst, zeros_init, n, e, 200)
    p0 = partials[0]
    p1 = partials[1]
    segg = _sc_seg_gather(p0, p1, dstn, np_, mr)

    return _run_p3(rows[:n], segg[:n], p0, p1, msgg[:n], node_inp,
                   WMk, Wak, Wquery, bquery, Wkey, bkey)


# speedup vs baseline: 14.7233x; 1.4147x over previous
import functools
import math

import jax
import jax.numpy as jnp
from jax import lax
from jax.experimental import pallas as pl
from jax.experimental.pallas import tpu as pltpu
from jax.experimental.pallas import tpu_sc as plsc

_NC = 2
_NS = 16
_NW = _NC * _NS


def _vgather(v, idx):
    return lax.gather(
        v, idx[:, None],
        lax.GatherDimensionNumbers(offset_dims=(), collapsed_slice_dims=(0,),
                                   start_index_map=(0,)),
        slice_sizes=(1,), mode=lax.GatherScatterMode.PROMISE_IN_BOUNDS)


def _block_diag(w):
    h, d, _ = w.shape
    eye = jnp.eye(h, dtype=w.dtype)
    out = jnp.einsum('hij,hg->higj', w, eye).reshape(h * d, h * d)
    return out


def _p1_body(x_ref, t_ref,
             wq0, wq1, bq0, bq1,
             wk0, wk1, bk0, bk1,
             wv0, wv1, bv0, bv1,
             bda0, bda1, bdm0, bdm1,
             q_out, k2_out, v2_out):
    x = x_ref[...]
    m0 = t_ref[...] == 0.0
    dot = functools.partial(jnp.dot, preferred_element_type=jnp.float32)
    q = jnp.where(m0, dot(x, wq0[...]) + bq0[...], dot(x, wq1[...]) + bq1[...])
    k = jnp.where(m0, dot(x, wk0[...]) + bk0[...], dot(x, wk1[...]) + bk1[...])
    v = jnp.where(m0, dot(x, wv0[...]) + bv0[...], dot(x, wv1[...]) + bv1[...])
    q_out[...] = q
    k2_out[:, 0, :] = dot(k, bda0[...])
    k2_out[:, 1, :] = dot(k, bda1[...])
    v2_out[:, 0, :] = dot(v, bdm0[...])
    v2_out[:, 1, :] = dot(v, bdm1[...])


def _run_p1(x, tf, wq, bq, wk, bk, wv, bv, bda, bdm):
    n, d = x.shape
    blk = 400
    grid = n // blk
    full = pl.BlockSpec((d, d), lambda i: (0, 0))
    fullb = pl.BlockSpec((1, d), lambda i: (0, 0))
    return pl.pallas_call(
        _p1_body,
        grid=(grid,),
        in_specs=[
            pl.BlockSpec((blk, d), lambda i: (i, 0)),
            pl.BlockSpec((blk, 1), lambda i: (i, 0)),
            full, full, fullb, fullb,
            full, full, fullb, fullb,
            full, full, fullb, fullb,
            full, full, full, full,
        ],
        out_specs=[
            pl.BlockSpec((blk, d), lambda i: (i, 0)),
            pl.BlockSpec((blk, 2, d), lambda i: (i, 0, 0)),
            pl.BlockSpec((blk, 2, d), lambda i: (i, 0, 0)),
        ],
        out_shape=[
            jax.ShapeDtypeStruct((n, d), jnp.float32),
            jax.ShapeDtypeStruct((n, 2, d), jnp.float32),
            jax.ShapeDtypeStruct((n, 2, d), jnp.float32),
        ],
    )(x, tf,
      wq[0], wq[1], bq[0][None], bq[1][None],
      wk[0], wk[1], bk[0][None], bk[1][None],
      wv[0], wv[1], bv[0][None], bv[1][None],
      bda[0], bda[1], bdm[0], bdm[1])


def _sc_fused(qh, k2h, v2h, dst, dst8, soff, k2i, mi, zeros_init,
              n, e, c, mr, np_):
    ew = e // _NW
    nch = ew // c
    nr = n // 8
    d = qh.shape[1]
    mesh = plsc.VectorSubcoreMesh(core_axis_name="c", subcore_axis_name="s")

    @functools.partial(
        pl.kernel, mesh=mesh,
        out_type=[
            jax.ShapeDtypeStruct((n, 16), jnp.float32),
            jax.ShapeDtypeStruct((2, nr, d), jnp.float32),
            jax.ShapeDtypeStruct((np_, d), jnp.float32),
        ],
        scratch_types=[
            pltpu.VMEM((c,), jnp.int32),
            pltpu.VMEM((c,), jnp.int32),
            pltpu.VMEM((c + 16,), jnp.int32),
            pltpu.VMEM((c,), jnp.int32),
            pltpu.VMEM((c, d), jnp.float32),
            pltpu.VMEM((c, d), jnp.float32),
            pltpu.VMEM((c, d), jnp.float32),
            pltpu.VMEM((c, 16), jnp.float32),
            pltpu.VMEM((mr,), jnp.int32),
            pltpu.VMEM((mr, d), jnp.float32),
            pltpu.VMEM_SHARED((nr, d), jnp.float32),
            pltpu.SemaphoreType.DMA,
        ],
    )
    def k(qh_h, k2h_h, v2h_h, dst_h, dst8_h, soff_h, k2i_h, mi_h, zero_h,
          rows_o, part_o, mg_o,
          dstb, d8b, sob, k2b, bq, bk2, rb, rb16, mib, mb, acc, sem):
        cid = lax.axis_index("c")
        sid = lax.axis_index("s")

        @pl.when(sid == 0)
        def _():
            pltpu.sync_copy(zero_h, acc)

        plsc.subcore_barrier()

        wid = sid * _NC + cid
        iota = lax.iota(jnp.int32, 16)
        zv = jnp.zeros((16,), jnp.float32)

        def chunk(i, carry):
            base = wid * ew + i * c
            pltpu.sync_copy(dst_h.at[pl.ds(base, c)], dstb)
            pltpu.sync_copy(dst8_h.at[pl.ds(base, c)], d8b)
            pltpu.sync_copy(soff_h.at[pl.ds(base, c)], sob.at[pl.ds(0, c)])
            pltpu.sync_copy(k2i_h.at[pl.ds(base, c)], k2b)
            pltpu.async_copy(qh_h.at[dstb], bq, sem).wait()
            pltpu.async_copy(k2h_h.at[k2b], bk2, sem).wait()

            def edge(j, carry2):
                accv = jnp.zeros((16,), jnp.float32)
                for h in range(8):
                    qv = bq[j, pl.ds(h * 16, 16)]
                    kv = bk2[j, pl.ds(h * 16, 16)]
                    s = qv * kv
                    for sh in (8, 4, 2, 1):
                        s = s + _vgather(s, jnp.bitwise_xor(iota, sh))
                    accv = jnp.where(iota == h, s, accv)
                ex = jnp.exp(accv)
                first = jnp.where(iota < 8, ex,
                                  jnp.where(iota == 8, 1.0, 0.0))
                rb16[j, :] = first
                so = sob[pl.ds(j, 16)][0]
                for slot in range(8):
                    rb[j, pl.ds(slot * 16, 16)] = jnp.where(
                        so == slot * 16, first, zv)
                return carry2

            lax.fori_loop(0, c, edge, 0)
            pltpu.sync_copy(rb, acc.at[d8b], add=True)

            @pl.when(base < n)
            def _():
                pltpu.sync_copy(rb16, rows_o.at[pl.ds(base, c)])

            return carry

        lax.fori_loop(0, nch, chunk, 0)

        mbase = wid * mr
        pltpu.sync_copy(mi_h.at[pl.ds(mbase, mr)], mib)
        pltpu.async_copy(v2h_h.at[mib], mb, sem).wait()
        pltpu.sync_copy(mb, mg_o.at[pl.ds(mbase, mr)])

        plsc.subcore_barrier()

        @pl.when(sid == 0)
        def _():
            pltpu.sync_copy(acc, part_o.at[cid])

    return k(qh, k2h, v2h, dst, dst8, soff, k2i, mi, zeros_init)


def _sc_seg_gather(p0, p1, dstn, np_, mr):
    mesh = plsc.VectorSubcoreMesh(core_axis_name="c", subcore_axis_name="s")

    @functools.partial(
        pl.kernel, mesh=mesh,
        out_type=jax.ShapeDtypeStruct((np_, 128), jnp.float32),
        scratch_types=[
            pltpu.VMEM((mr,), jnp.int32),
            pltpu.VMEM((mr, 128), jnp.float32),
            pltpu.VMEM((mr, 128), jnp.float32),
            pltpu.SemaphoreType.DMA,
        ],
    )
    def k(p0_h, p1_h, dstn_h, seg_o, ib, b0, b1, sem):
        wid = lax.axis_index("s") * _NC + lax.axis_index("c")
        base = wid * mr
        pltpu.sync_copy(dstn_h.at[pl.ds(base, mr)], ib)
        pltpu.async_copy(p0_h.at[ib], b0, sem).wait()
        pltpu.async_copy(p1_h.at[ib], b1, sem).wait()

        def add1(i, carry):
            b0[i, :] = b0[i, :] + b1[i, :]
            return carry

        lax.fori_loop(0, mr, add1, 0)
        pltpu.sync_copy(b0, seg_o.at[pl.ds(base, mr)])

    return k(p0, p1, dstn)


def _p3_body(ex_ref, sg_ref, dm_ref, p0_ref, p1_ref, msg_ref, x_ref,
             wm0, wm1, wm2, wa0, wa1, wa2, wq, bq, wkey, bkey, out_ref):
    dot = functools.partial(jnp.dot, preferred_element_type=jnp.float32)
    ex = ex_ref[...][:, 0:8]
    sg = sg_ref[...]
    dm = dm_ref[...]
    bsz = sg.shape[0]
    seg16 = jnp.zeros((bsz, 16), jnp.float32)
    for s in range(8):
        seg16 = seg16 + jnp.where(dm == float(s),
                                  sg[:, s * 16:(s + 1) * 16], 0.0)
    ss = seg16[:, 0:8]
    att = ex / (ss + 1e-16)
    deg = p0_ref[...][:, 8:9] + p1_ref[...][:, 8:9]
    m = msg_ref[...]
    b = m.shape[0]
    att128 = jnp.concatenate(
        [jnp.broadcast_to(att[:, h:h + 1], (b, 16)) for h in range(8)], axis=1)
    r1 = m * att128
    r2 = m * m * att128
    r3 = m * m * m * att128
    agg1 = deg * r1
    agg2 = deg * r2
    agg3 = deg * r3
    agg3 = jnp.sign(agg3) * jnp.exp(
        jnp.log(jnp.abs(agg3) + 1e-18) * (1.0 / 3.0))
    g1 = dot(agg1, wm0[...])
    g2 = dot(agg2, wm1[...])
    g3 = dot(agg3, wm2[...])
    qn = dot(x_ref[...], wq[...]) + bq[...]
    res = jnp.zeros((b, m.shape[1]), jnp.float32)
    for g, wa in ((g1, wa0), (g2, wa1), (g3, wa2)):
        front = dot(qn, wa[...])
        tail = dot(g, wkey[...]) + bkey[...]
        s = jnp.sum(front * tail, axis=1, keepdims=True)
        score = 1.0 / (1.0 + jnp.exp(-s))
        res = res + score * g
    out_ref[...] = res


def _run_p3(exn, segn, dm, p0, p1, msgn, x, wmk, wak, wq, bq, wkey, bkey):
    n, d = x.shape
    blk = 400
    grid = n // blk
    full = pl.BlockSpec((d, d), lambda i: (0, 0))
    fullb = pl.BlockSpec((1, d), lambda i: (0, 0))
    row16 = pl.BlockSpec((blk, 16), lambda i: (i, 0))
    return pl.pallas_call(
        _p3_body,
        grid=(grid,),
        in_specs=[
            row16,
            pl.BlockSpec((blk, d), lambda i: (i, 0)),
            pl.BlockSpec((blk, 1), lambda i: (i, 0)),
            row16, row16,
            pl.BlockSpec((blk, d), lambda i: (i, 0)),
            pl.BlockSpec((blk, d), lambda i: (i, 0)),
            full, full, full, full, full, full,
            full, fullb, full, fullb,
        ],
        out_specs=pl.BlockSpec((blk, d), lambda i: (i, 0)),
        out_shape=jax.ShapeDtypeStruct((n, d), jnp.float32),
    )(exn, segn, dm, p0, p1, msgn, x,
      wmk[0], wmk[1], wmk[2], wak[0], wak[1], wak[2],
      wq, bq[None], wkey, bkey[None])


def kernel(node_inp, node_type, edge_index, edge_type, edge_time,
           Wk, bk, Wq, bq, Wv, bv, rel_pri, rel_att, rel_msg,
           WMk, Wak, Wquery, bquery, Wkey, bkey):
    n, in_dim = node_inp.shape
    e = edge_index.shape[1]
    num_rel, n_heads, d_k, _ = rel_att.shape
    sqrt_dk = math.sqrt(d_k)

    src = edge_index[0].astype(jnp.int32)
    dst = edge_index[1].astype(jnp.int32)
    et = edge_type.astype(jnp.int32)
    k2idx = src * num_rel + et

    tf = node_type.astype(jnp.float32).reshape(n, 1)
    bda = [_block_diag(rel_att[r] * (rel_pri[r][:, None, None] / sqrt_dk))
           for r in range(num_rel)]
    bdm = [_block_diag(rel_msg[r]) for r in range(num_rel)]

    qh, k2, v2 = _run_p1(node_inp, tf, Wq, bq, Wk, bk, Wv, bv, bda, bdm)
    k2 = k2.reshape(n * 2, in_dim)
    v2 = v2.reshape(n * 2, in_dim)

    mr = ((n + _NW - 1) // _NW + 7) // 8 * 8
    np_ = _NW * mr
    pad = np_ - n
    mi = jnp.concatenate([k2idx[:n], jnp.zeros((pad,), jnp.int32)])
    dstn = jnp.concatenate([dst[:n], jnp.zeros((pad,), jnp.int32)])

    dst8 = dst // 8
    soff = (dst % 8) * 16
    dstn8 = jnp.concatenate([dst8[:n], jnp.zeros((pad,), jnp.int32)])
    dm = (dst[:n] % 8).astype(jnp.float32).reshape(n, 1)
    zeros_init = jnp.zeros((n // 8, 128), jnp.float32)
    rows16, partials, msgg = _sc_fused(qh, k2, v2, dst, dst8, soff, k2idx,
                                       mi, zeros_init, n, e, 80, mr, np_)
    p0r = partials[0].reshape(n, 16)
    p1r = partials[1].reshape(n, 16)
    segg = _sc_seg_gather(partials[0], partials[1], dstn8, np_, mr)

    return _run_p3(rows16, segg[:n], dm, p0r, p1r, msgg[:n], node_inp,
                   WMk, Wak, Wquery, bquery, Wkey, bkey)


# speedup vs baseline: 16.1103x; 1.0942x over previous
import functools
import math

import jax
import jax.numpy as jnp
from jax import lax
from jax.experimental import pallas as pl
from jax.experimental.pallas import tpu as pltpu
from jax.experimental.pallas import tpu_sc as plsc

_NC = 2
_NS = 16
_NW = _NC * _NS


def _vgather(v, idx):
    return lax.gather(
        v, idx[:, None],
        lax.GatherDimensionNumbers(offset_dims=(), collapsed_slice_dims=(0,),
                                   start_index_map=(0,)),
        slice_sizes=(1,), mode=lax.GatherScatterMode.PROMISE_IN_BOUNDS)


def _block_diag(w):
    h, d, _ = w.shape
    eye = jnp.eye(h, dtype=w.dtype)
    out = jnp.einsum('hij,hg->higj', w, eye).reshape(h * d, h * d)
    return out


def _p1_body(x_ref, t_ref,
             wq0, wq1, bq0, bq1,
             wk0, wk1, bk0, bk1,
             wv0, wv1, bv0, bv1,
             bda0, bda1, bdm0, bdm1,
             q_out, k2_out, v2_out):
    x = x_ref[...]
    m0 = t_ref[...] == 0.0
    dot = functools.partial(jnp.dot, preferred_element_type=jnp.float32)
    q = jnp.where(m0, dot(x, wq0[...]) + bq0[...], dot(x, wq1[...]) + bq1[...])
    k = jnp.where(m0, dot(x, wk0[...]) + bk0[...], dot(x, wk1[...]) + bk1[...])
    v = jnp.where(m0, dot(x, wv0[...]) + bv0[...], dot(x, wv1[...]) + bv1[...])
    q_out[...] = q
    k2_out[:, 0, :] = dot(k, bda0[...])
    k2_out[:, 1, :] = dot(k, bda1[...])
    v2_out[:, 0, :] = dot(v, bdm0[...])
    v2_out[:, 1, :] = dot(v, bdm1[...])


def _run_p1(x, tf, wq, bq, wk, bk, wv, bv, bda, bdm):
    n, d = x.shape
    blk = 400
    grid = n // blk
    full = pl.BlockSpec((d, d), lambda i: (0, 0))
    fullb = pl.BlockSpec((1, d), lambda i: (0, 0))
    return pl.pallas_call(
        _p1_body,
        grid=(grid,),
        in_specs=[
            pl.BlockSpec((blk, d), lambda i: (i, 0)),
            pl.BlockSpec((blk, 1), lambda i: (i, 0)),
            full, full, fullb, fullb,
            full, full, fullb, fullb,
            full, full, fullb, fullb,
            full, full, full, full,
        ],
        out_specs=[
            pl.BlockSpec((blk, d), lambda i: (i, 0)),
            pl.BlockSpec((blk, 2, d), lambda i: (i, 0, 0)),
            pl.BlockSpec((blk, 2, d), lambda i: (i, 0, 0)),
        ],
        out_shape=[
            jax.ShapeDtypeStruct((n, d), jnp.float32),
            jax.ShapeDtypeStruct((n, 2, d), jnp.float32),
            jax.ShapeDtypeStruct((n, 2, d), jnp.float32),
        ],
    )(x, tf,
      wq[0], wq[1], bq[0][None], bq[1][None],
      wk[0], wk[1], bk[0][None], bk[1][None],
      wv[0], wv[1], bv[0][None], bv[1][None],
      bda[0], bda[1], bdm[0], bdm[1])


def _sc_fused(qh, k2h, v2h, dst, dst8, soff, k2i, mi, zeros_init,
              n, e, c, mr, np_):
    ew = e // _NW
    nch = ew // c
    nr = n // 8
    d = qh.shape[1]
    mesh = plsc.VectorSubcoreMesh(core_axis_name="c", subcore_axis_name="s")

    @functools.partial(
        pl.kernel, mesh=mesh,
        out_type=[
            jax.ShapeDtypeStruct((n, 16), jnp.float32),
            jax.ShapeDtypeStruct((2, nr, d), jnp.float32),
            jax.ShapeDtypeStruct((np_, d), jnp.float32),
        ],
        scratch_types=[
            pltpu.VMEM((c,), jnp.int32),
            pltpu.VMEM((c,), jnp.int32),
            pltpu.VMEM((c + 16,), jnp.int32),
            pltpu.VMEM((c,), jnp.int32),
            pltpu.VMEM((c, d), jnp.float32),
            pltpu.VMEM((c, d), jnp.float32),
            pltpu.VMEM((c, d), jnp.float32),
            pltpu.VMEM((c, 16), jnp.float32),
            pltpu.VMEM((mr,), jnp.int32),
            pltpu.VMEM((mr, d), jnp.float32),
            pltpu.VMEM_SHARED((nr, d), jnp.float32),
            pltpu.SemaphoreType.DMA,
        ],
    )
    def k(qh_h, k2h_h, v2h_h, dst_h, dst8_h, soff_h, k2i_h, mi_h, zero_h,
          rows_o, part_o, mg_o,
          dstb, d8b, sob, k2b, bq, bk2, rb, rb16, mib, mb, acc, sem):
        cid = lax.axis_index("c")
        sid = lax.axis_index("s")

        @pl.when(sid == 0)
        def _():
            pltpu.sync_copy(zero_h, acc)

        plsc.subcore_barrier()

        wid = sid * _NC + cid
        iota = lax.iota(jnp.int32, 16)
        zv = jnp.zeros((16,), jnp.float32)

        def chunk(i, carry):
            base = wid * ew + i * c
            pltpu.sync_copy(dst_h.at[pl.ds(base, c)], dstb)
            pltpu.sync_copy(dst8_h.at[pl.ds(base, c)], d8b)
            pltpu.sync_copy(soff_h.at[pl.ds(base, c)], sob.at[pl.ds(0, c)])
            pltpu.sync_copy(k2i_h.at[pl.ds(base, c)], k2b)
            dq = pltpu.async_copy(qh_h.at[dstb], bq, sem)
            dk = pltpu.async_copy(k2h_h.at[k2b], bk2, sem)
            dq.wait()
            dk.wait()

            def edge(j, carry2):
                accv = jnp.zeros((16,), jnp.float32)
                for h in range(8):
                    qv = bq[j, pl.ds(h * 16, 16)]
                    kv = bk2[j, pl.ds(h * 16, 16)]
                    s = qv * kv
                    for sh in (8, 4, 2, 1):
                        s = s + _vgather(s, jnp.bitwise_xor(iota, sh))
                    accv = jnp.where(iota == h, s, accv)
                ex = jnp.exp(accv)
                first = jnp.where(iota < 8, ex,
                                  jnp.where(iota == 8, 1.0, 0.0))
                rb16[j, :] = first
                so = sob[pl.ds(j, 16)][0]
                for slot in range(8):
                    rb[j, pl.ds(slot * 16, 16)] = jnp.where(
                        so == slot * 16, first, zv)
                return carry2

            lax.fori_loop(0, c, edge, 0)
            pltpu.sync_copy(rb, acc.at[d8b], add=True)

            @pl.when(base < n)
            def _():
                pltpu.sync_copy(rb16, rows_o.at[pl.ds(base, c)])

            return carry

        lax.fori_loop(0, nch, chunk, 0)

        mbase = wid * mr
        pltpu.sync_copy(mi_h.at[pl.ds(mbase, mr)], mib)
        pltpu.async_copy(v2h_h.at[mib], mb, sem).wait()
        pltpu.sync_copy(mb, mg_o.at[pl.ds(mbase, mr)])

        plsc.subcore_barrier()

        @pl.when(sid == 0)
        def _():
            pltpu.sync_copy(acc, part_o.at[cid])

    return k(qh, k2h, v2h, dst, dst8, soff, k2i, mi, zeros_init)


def _sc_seg_gather(p0, p1, dstn, np_, mr):
    mesh = plsc.VectorSubcoreMesh(core_axis_name="c", subcore_axis_name="s")

    @functools.partial(
        pl.kernel, mesh=mesh,
        out_type=jax.ShapeDtypeStruct((np_, 128), jnp.float32),
        scratch_types=[
            pltpu.VMEM((mr,), jnp.int32),
            pltpu.VMEM((mr, 128), jnp.float32),
            pltpu.VMEM((mr, 128), jnp.float32),
            pltpu.SemaphoreType.DMA,
        ],
    )
    def k(p0_h, p1_h, dstn_h, seg_o, ib, b0, b1, sem):
        wid = lax.axis_index("s") * _NC + lax.axis_index("c")
        base = wid * mr
        pltpu.sync_copy(dstn_h.at[pl.ds(base, mr)], ib)
        pltpu.async_copy(p0_h.at[ib], b0, sem).wait()
        pltpu.async_copy(p1_h.at[ib], b1, sem).wait()

        def add1(i, carry):
            b0[i, :] = b0[i, :] + b1[i, :]
            return carry

        lax.fori_loop(0, mr, add1, 0)
        pltpu.sync_copy(b0, seg_o.at[pl.ds(base, mr)])

    return k(p0, p1, dstn)


def _p3_body(ex_ref, sg_ref, dm_ref, p0_ref, p1_ref, msg_ref, x_ref,
             wm0, wm1, wm2, wa0, wa1, wa2, wq, bq, wkey, bkey, out_ref):
    dot = functools.partial(jnp.dot, preferred_element_type=jnp.float32)
    ex = ex_ref[...][:, 0:8]
    sg = sg_ref[...]
    dm = dm_ref[...]
    bsz = sg.shape[0]
    seg16 = jnp.zeros((bsz, 16), jnp.float32)
    for s in range(8):
        seg16 = seg16 + jnp.where(dm == float(s),
                                  sg[:, s * 16:(s + 1) * 16], 0.0)
    ss = seg16[:, 0:8]
    att = ex / (ss + 1e-16)
    deg = p0_ref[...][:, 8:9] + p1_ref[...][:, 8:9]
    m = msg_ref[...]
    b = m.shape[0]
    att128 = jnp.concatenate(
        [jnp.broadcast_to(att[:, h:h + 1], (b, 16)) for h in range(8)], axis=1)
    r1 = m * att128
    r2 = m * m * att128
    r3 = m * m * m * att128
    agg1 = deg * r1
    agg2 = deg * r2
    agg3 = deg * r3
    agg3 = jnp.sign(agg3) * jnp.exp(
        jnp.log(jnp.abs(agg3) + 1e-18) * (1.0 / 3.0))
    g1 = dot(agg1, wm0[...])
    g2 = dot(agg2, wm1[...])
    g3 = dot(agg3, wm2[...])
    qn = dot(x_ref[...], wq[...]) + bq[...]
    res = jnp.zeros((b, m.shape[1]), jnp.float32)
    for g, wa in ((g1, wa0), (g2, wa1), (g3, wa2)):
        front = dot(qn, wa[...])
        tail = dot(g, wkey[...]) + bkey[...]
        s = jnp.sum(front * tail, axis=1, keepdims=True)
        score = 1.0 / (1.0 + jnp.exp(-s))
        res = res + score * g
    out_ref[...] = res


def _run_p3(exn, segn, dm, p0, p1, msgn, x, wmk, wak, wq, bq, wkey, bkey):
    n, d = x.shape
    blk = 400
    grid = n // blk
    full = pl.BlockSpec((d, d), lambda i: (0, 0))
    fullb = pl.BlockSpec((1, d), lambda i: (0, 0))
    row16 = pl.BlockSpec((blk, 16), lambda i: (i, 0))
    return pl.pallas_call(
        _p3_body,
        grid=(grid,),
        in_specs=[
            row16,
            pl.BlockSpec((blk, d), lambda i: (i, 0)),
            pl.BlockSpec((blk, 1), lambda i: (i, 0)),
            row16, row16,
            pl.BlockSpec((blk, d), lambda i: (i, 0)),
            pl.BlockSpec((blk, d), lambda i: (i, 0)),
            full, full, full, full, full, full,
            full, fullb, full, fullb,
        ],
        out_specs=pl.BlockSpec((blk, d), lambda i: (i, 0)),
        out_shape=jax.ShapeDtypeStruct((n, d), jnp.float32),
    )(exn, segn, dm, p0, p1, msgn, x,
      wmk[0], wmk[1], wmk[2], wak[0], wak[1], wak[2],
      wq, bq[None], wkey, bkey[None])


def kernel(node_inp, node_type, edge_index, edge_type, edge_time,
           Wk, bk, Wq, bq, Wv, bv, rel_pri, rel_att, rel_msg,
           WMk, Wak, Wquery, bquery, Wkey, bkey):
    n, in_dim = node_inp.shape
    e = edge_index.shape[1]
    num_rel, n_heads, d_k, _ = rel_att.shape
    sqrt_dk = math.sqrt(d_k)

    src = edge_index[0].astype(jnp.int32)
    dst = edge_index[1].astype(jnp.int32)
    et = edge_type.astype(jnp.int32)
    k2idx = src * num_rel + et

    tf = node_type.astype(jnp.float32).reshape(n, 1)
    bda = [_block_diag(rel_att[r] * (rel_pri[r][:, None, None] / sqrt_dk))
           for r in range(num_rel)]
    bdm = [_block_diag(rel_msg[r]) for r in range(num_rel)]

    qh, k2, v2 = _run_p1(node_inp, tf, Wq, bq, Wk, bk, Wv, bv, bda, bdm)
    k2 = k2.reshape(n * 2, in_dim)
    v2 = v2.reshape(n * 2, in_dim)

    mr = ((n + _NW - 1) // _NW + 7) // 8 * 8
    np_ = _NW * mr
    pad = np_ - n
    mi = jnp.concatenate([k2idx[:n], jnp.zeros((pad,), jnp.int32)])

    dst8 = dst // 8
    soff = (dst % 8) * 16
    dstn8 = jnp.concatenate([dst8[:n], jnp.zeros((pad,), jnp.int32)])
    dm = (dst[:n] % 8).astype(jnp.float32).reshape(n, 1)
    zeros_init = jnp.zeros((n // 8, 128), jnp.float32)
    rows16, partials, msgg = _sc_fused(qh, k2, v2, dst, dst8, soff, k2idx,
                                       mi, zeros_init, n, e, 80, mr, np_)
    p0r = partials[0].reshape(n, 16)
    p1r = partials[1].reshape(n, 16)
    segg = _sc_seg_gather(partials[0], partials[1], dstn8, np_, mr)

    return _run_p3(rows16, segg[:n], dm, p0r, p1r, msgg[:n], node_inp,
                   WMk, Wak, Wquery, bquery, Wkey, bkey)
